# baseline (speedup 1.0000x reference)
import functools

import jax
import jax.numpy as jnp
from jax import lax
from jax.experimental import pallas as pl
from jax.experimental.pallas import tpu as pltpu
from jax.experimental.pallas import tpu_sc as plsc

BATCH = 16384
D = 64
NC = 2
NS = 16
L = 16
NW = NC * NS
BPW = BATCH // NW
GC = 128
NG = BPW // GC


@functools.partial(
    pl.kernel,
    mesh=plsc.VectorSubcoreMesh(core_axis_name="c", subcore_axis_name="s"),
    out_type=jax.ShapeDtypeStruct((BATCH,), jnp.float32),
    compiler_params=pltpu.CompilerParams(use_tc_tiling_on_sc=False, needs_layout_passes=False),
    scratch_types=[
        pltpu.VMEM((NG, GC), jnp.int32),
        pltpu.VMEM((NG, GC), jnp.int32),
        pltpu.VMEM((BPW, D), jnp.float32),
        pltpu.VMEM((BPW, D), jnp.float32),
        pltpu.VMEM((BPW,), jnp.float32),
        pltpu.SemaphoreType.DMA,
    ],
)
def _pointmf_sc(user_hbm, item_hbm, eu_hbm, ei_hbm, out_hbm,
                uidx, iidx, urows, irows, outv, sem):
    wid = lax.axis_index("c") * NS + lax.axis_index("s")
    base = wid * BPW

    for g in range(NG):
        pltpu.sync_copy(user_hbm.at[pl.ds(base + g * GC, GC)], uidx.at[g])
        pltpu.sync_copy(item_hbm.at[pl.ds(base + g * GC, GC)], iidx.at[g])

    copies = []
    for g in range(NG):
        copies.append(
            pltpu.async_copy(eu_hbm.at[uidx.at[g]],
                             urows.at[pl.ds(g * GC, GC)], sem))
        copies.append(
            pltpu.async_copy(ei_hbm.at[iidx.at[g]],
                             irows.at[pl.ds(g * GC, GC)], sem))
    for c in copies:
        c.wait()

    lane = lax.broadcasted_iota(jnp.int32, (L,), 0)

    def chunk(ci, carry):
        rows = ci * L + lane
        accs = [jnp.zeros((L,), jnp.float32) for _ in range(4)]
        for d in range(D):
            col = jnp.full((L,), d, jnp.int32)
            u = plsc.load_gather(urows, [rows, col])
            v = plsc.load_gather(irows, [rows, col])
            accs[d % 4] = accs[d % 4] + u * v
        acc = (accs[0] + accs[1]) + (accs[2] + accs[3])
        plsc.store_scatter(outv, [rows], acc)
        return carry

    lax.fori_loop(0, BPW // L, chunk, 0)

    pltpu.sync_copy(outv, out_hbm.at[pl.ds(base, BPW)])


def kernel(user, item, embed_user, embed_item):
    return _pointmf_sc(user, item, embed_user, embed_item)
